# baseline (speedup 1.0000x reference)
import jax
import jax.numpy as jnp
from jax.experimental import pallas as pl


_BLOCK_ROWS = 888


def _ew_kernel(x_ref, o_ref):
    x = x_ref[...]
    o_ref[...] = jnp.where(x > -1.0, jnp.float32(3.0), x + 1.0)


def kernel(x):
    b, m, n = x.shape
    x2 = x.reshape(b * m, n)
    rows = b * m
    out = pl.pallas_call(
        _ew_kernel,
        grid=(pl.cdiv(rows, _BLOCK_ROWS),),
        in_specs=[pl.BlockSpec((_BLOCK_ROWS, n), lambda i: (i, 0))],
        out_specs=pl.BlockSpec((_BLOCK_ROWS, n), lambda i: (i, 0)),
        out_shape=jax.ShapeDtypeStruct((rows, n), x.dtype),
    )(x2)
    return out.reshape(b, m, n)
